# baseline (speedup 1.0000x reference)
import functools

import jax
import jax.numpy as jnp
from jax.experimental import pallas as pl
from jax.experimental.pallas import tpu as pltpu

N_TOK = 16384
D = 4096
E = 64
K_TOP = 8
COEF = 0.01

BM = 1024
BN = 1024
BK = 512

M_T = N_TOK // BM
N_T = D // BN
K_T = D // BK

def _dot_t(a, b):
    return jax.lax.dot_general(
        a.astype(jnp.bfloat16), b.astype(jnp.bfloat16),
        (((1,), (1,)), ((), ())),
        preferred_element_type=jnp.float32)


def _router_kernel(x_ref, w1_ref, b1_ref, w2_ref, b2_ref,
                   topw_ref, topi_ref, loss_ref,
                   acc_ref, logits_ref, psum_ref):
    m = pl.program_id(0)
    n = pl.program_id(1)
    k = pl.program_id(2)

    @pl.when(k == 0)
    def _():
        acc_ref[...] = jnp.zeros_like(acc_ref)

    acc_ref[...] += _dot_t(x_ref[...], w1_ref[...])

    @pl.when(k == K_T - 1)
    def _():
        h = acc_ref[...] + b1_ref[...]
        h = h * jax.nn.sigmoid(h)

        @pl.when(n == 0)
        def _():
            logits_ref[...] = jnp.zeros_like(logits_ref)

        logits_ref[...] += _dot_t(h, w2_ref[...])

        @pl.when(n == N_T - 1)
        def _():
            lg = logits_ref[...] + b2_ref[...]
            lanes = jax.lax.broadcasted_iota(jnp.int32, (BM, E), 1)
            work = lg
            vals = []
            idxs = []
            for _j in range(K_TOP):
                mx = jnp.max(work, axis=1, keepdims=True)
                hit = work >= mx
                am = jnp.min(jnp.where(hit, lanes, E), axis=1,
                             keepdims=True)
                vals.append(mx)
                idxs.append(am)
                work = jnp.where(lanes == am, -jnp.inf, work)
            tv = jnp.concatenate(vals, axis=1)
            ti = jnp.concatenate(idxs, axis=1)

            ew = jnp.exp(tv - tv[:, 0:1])
            topw_ref[...] = ew / jnp.sum(ew, axis=1, keepdims=True)
            topi_ref[...] = ti

            ep = jnp.exp(lg - tv[:, 0:1])
            p = ep / jnp.sum(ep, axis=1, keepdims=True)
            part = jnp.sum(p, axis=0, keepdims=True)

            @pl.when(m == 0)
            def _():
                psum_ref[...] = jnp.zeros_like(psum_ref)

            psum_ref[0:1, :] += part

            @pl.when(m == M_T - 1)
            def _():
                avg = psum_ref[0:1, :] / N_TOK
                diff = avg - (1.0 / E)
                loss_ref[...] = (COEF / E) * jnp.sum(
                    diff * diff, axis=1, keepdims=True)


@jax.jit
def kernel(x, W1, b1, W2, b2):
    b1r = b1.reshape(1, D)
    b2r = b2.reshape(1, E)
    grid = (M_T, N_T, K_T)
    topw, topi, loss = pl.pallas_call(
        _router_kernel,
        grid=grid,
        in_specs=[
            pl.BlockSpec((BM, BK), lambda m, n, k: (m, k)),
            pl.BlockSpec((BN, BK), lambda m, n, k: (n, k)),
            pl.BlockSpec((1, BN), lambda m, n, k: (0, n)),
            pl.BlockSpec((E, BN), lambda m, n, k: (0, n)),
            pl.BlockSpec((1, E), lambda m, n, k: (0, 0)),
        ],
        out_specs=[
            pl.BlockSpec((BM, K_TOP), lambda m, n, k: (m, 0)),
            pl.BlockSpec((BM, K_TOP), lambda m, n, k: (m, 0)),
            pl.BlockSpec((1, 1), lambda m, n, k: (0, 0)),
        ],
        out_shape=[
            jax.ShapeDtypeStruct((N_TOK, K_TOP), jnp.float32),
            jax.ShapeDtypeStruct((N_TOK, K_TOP), jnp.int32),
            jax.ShapeDtypeStruct((1, 1), jnp.float32),
        ],
        scratch_shapes=[
            pltpu.VMEM((BM, BN), jnp.float32),
            pltpu.VMEM((BM, E), jnp.float32),
            pltpu.VMEM((8, E), jnp.float32),
        ],
        compiler_params=pltpu.CompilerParams(
            dimension_semantics=("arbitrary", "arbitrary", "arbitrary"),
        ),
    )(x, W1, b1r, W2, b2r)
    return topw, topi, loss.reshape(())


# speedup vs baseline: 1.3606x; 1.3606x over previous
import jax
import jax.numpy as jnp
from jax.experimental import pallas as pl
from jax.experimental.pallas import tpu as pltpu

N_TOK = 16384
D = 4096
E = 64
K_TOP = 8
COEF = 0.01

BM = 1024
BK = 256

M_T = N_TOK // BM
K_T = D // BK


def _dot_t(a, b):
    return jax.lax.dot_general(
        a.astype(jnp.bfloat16), b.astype(jnp.bfloat16),
        (((1,), (1,)), ((), ())),
        preferred_element_type=jnp.float32)


def _router_kernel(x_ref, w1_ref, b1_ref, w2_ref, b2_ref,
                   topw_ref, topi_ref, loss_ref,
                   acc_ref, psum_ref):
    m = pl.program_id(0)
    k = pl.program_id(1)

    @pl.when(k == 0)
    def _():
        acc_ref[...] = jnp.zeros_like(acc_ref)

    acc_ref[...] += _dot_t(x_ref[...], w1_ref[...])

    @pl.when(k == K_T - 1)
    def _():
        h = acc_ref[...] + b1_ref[...]
        h = h * jax.nn.sigmoid(h)
        lg = _dot_t(h, w2_ref[...]) + b2_ref[...]

        lanes = jax.lax.broadcasted_iota(jnp.int32, (BM, E), 1)
        work = lg
        vals = []
        idxs = []
        for _j in range(K_TOP):
            mx = jnp.max(work, axis=1, keepdims=True)
            hit = work >= mx
            am = jnp.min(jnp.where(hit, lanes, E), axis=1,
                         keepdims=True)
            vals.append(mx)
            idxs.append(am)
            work = jnp.where(lanes == am, -jnp.inf, work)
        tv = jnp.concatenate(vals, axis=1)
        ti = jnp.concatenate(idxs, axis=1)

        ew = jnp.exp(tv - tv[:, 0:1])
        topw_ref[...] = ew / jnp.sum(ew, axis=1, keepdims=True)
        topi_ref[...] = ti

        ep = jnp.exp(lg - tv[:, 0:1])
        p = ep / jnp.sum(ep, axis=1, keepdims=True)
        part = jnp.sum(p, axis=0, keepdims=True)

        @pl.when(m == 0)
        def _():
            psum_ref[...] = jnp.zeros_like(psum_ref)

        psum_ref[0:1, :] += part

        @pl.when(m == M_T - 1)
        def _():
            avg = psum_ref[0:1, :] / N_TOK
            diff = avg - (1.0 / E)
            loss_ref[...] = (COEF / E) * jnp.sum(
                diff * diff, axis=1, keepdims=True)


@jax.jit
def kernel(x, W1, b1, W2, b2):
    b1r = b1.reshape(1, D)
    b2r = b2.reshape(1, E)
    grid = (M_T, K_T)
    topw, topi, loss = pl.pallas_call(
        _router_kernel,
        grid=grid,
        in_specs=[
            pl.BlockSpec((BM, BK), lambda m, k: (m, k)),
            pl.BlockSpec((D, BK), lambda m, k: (0, k)),
            pl.BlockSpec((1, D), lambda m, k: (0, 0)),
            pl.BlockSpec((E, D), lambda m, k: (0, 0)),
            pl.BlockSpec((1, E), lambda m, k: (0, 0)),
        ],
        out_specs=[
            pl.BlockSpec((BM, K_TOP), lambda m, k: (m, 0)),
            pl.BlockSpec((BM, K_TOP), lambda m, k: (m, 0)),
            pl.BlockSpec((1, 1), lambda m, k: (0, 0)),
        ],
        out_shape=[
            jax.ShapeDtypeStruct((N_TOK, K_TOP), jnp.float32),
            jax.ShapeDtypeStruct((N_TOK, K_TOP), jnp.int32),
            jax.ShapeDtypeStruct((1, 1), jnp.float32),
        ],
        scratch_shapes=[
            pltpu.VMEM((BM, D), jnp.float32),
            pltpu.VMEM((8, E), jnp.float32),
        ],
        compiler_params=pltpu.CompilerParams(
            dimension_semantics=("arbitrary", "arbitrary"),
        ),
    )(x, W1, b1r, W2, b2r)
    return topw, topi, loss.reshape(())


# speedup vs baseline: 1.4399x; 1.0583x over previous
import jax
import jax.numpy as jnp
from jax.experimental import pallas as pl
from jax.experimental.pallas import tpu as pltpu

N_TOK = 16384
D = 4096
E = 64
K_TOP = 8
COEF = 0.01

BM = 1024
BN = 512

M_T = N_TOK // BM
N_T = D // BN


def _dot_t(a, b):
    return jax.lax.dot_general(
        a, b, (((1,), (1,)), ((), ())),
        preferred_element_type=jnp.float32)


def _router_kernel(x_ref, w1_ref, b1_ref, w2_ref, b2_ref,
                   topw_ref, topi_ref, loss_ref,
                   xb_ref, logits_ref, psum_ref):
    m = pl.program_id(0)
    n = pl.program_id(1)

    @pl.when(n == 0)
    def _():
        xb_ref[...] = x_ref[...].astype(jnp.bfloat16)

    h = _dot_t(xb_ref[...], w1_ref[...]) + b1_ref[...]
    h = h * jax.nn.sigmoid(h)
    part = _dot_t(h.astype(jnp.bfloat16), w2_ref[...])

    @pl.when(n == 0)
    def _():
        logits_ref[...] = part

    @pl.when(n > 0)
    def _():
        logits_ref[...] += part

    @pl.when(n == N_T - 1)
    def _():
        lg = logits_ref[...] + b2_ref[...]

        lanes = jax.lax.broadcasted_iota(jnp.int32, (BM, E), 1)
        work = lg
        vals = []
        idxs = []
        for _j in range(K_TOP):
            mx = jnp.max(work, axis=1, keepdims=True)
            hit = work >= mx
            am = jnp.min(jnp.where(hit, lanes, E), axis=1,
                         keepdims=True)
            vals.append(mx)
            idxs.append(am)
            work = jnp.where(lanes == am, -jnp.inf, work)
        tv = jnp.concatenate(vals, axis=1)
        ti = jnp.concatenate(idxs, axis=1)

        ew = jnp.exp(tv - tv[:, 0:1])
        topw_ref[...] = ew / jnp.sum(ew, axis=1, keepdims=True)
        topi_ref[...] = ti

        ep = jnp.exp(lg - tv[:, 0:1])
        p = ep / jnp.sum(ep, axis=1, keepdims=True)
        psum_part = jnp.sum(p, axis=0, keepdims=True)

        @pl.when(m == 0)
        def _():
            psum_ref[...] = jnp.zeros_like(psum_ref)

        psum_ref[0:1, :] += psum_part

        @pl.when(m == M_T - 1)
        def _():
            avg = psum_ref[0:1, :] / N_TOK
            diff = avg - (1.0 / E)
            loss_ref[...] = (COEF / E) * jnp.sum(
                diff * diff, axis=1, keepdims=True)


@jax.jit
def kernel(x, W1, b1, W2, b2):
    W1b = W1.astype(jnp.bfloat16)
    W2b = W2.astype(jnp.bfloat16)
    b1r = b1.reshape(1, D)
    b2r = b2.reshape(1, E)
    grid = (M_T, N_T)
    topw, topi, loss = pl.pallas_call(
        _router_kernel,
        grid=grid,
        in_specs=[
            pl.BlockSpec((BM, D), lambda m, n: (m, 0)),
            pl.BlockSpec((BN, D), lambda m, n: (n, 0)),
            pl.BlockSpec((1, BN), lambda m, n: (0, n)),
            pl.BlockSpec((E, BN), lambda m, n: (0, n)),
            pl.BlockSpec((1, E), lambda m, n: (0, 0)),
        ],
        out_specs=[
            pl.BlockSpec((BM, K_TOP), lambda m, n: (m, 0)),
            pl.BlockSpec((BM, K_TOP), lambda m, n: (m, 0)),
            pl.BlockSpec((1, 1), lambda m, n: (0, 0)),
        ],
        out_shape=[
            jax.ShapeDtypeStruct((N_TOK, K_TOP), jnp.float32),
            jax.ShapeDtypeStruct((N_TOK, K_TOP), jnp.int32),
            jax.ShapeDtypeStruct((1, 1), jnp.float32),
        ],
        scratch_shapes=[
            pltpu.VMEM((BM, D), jnp.bfloat16),
            pltpu.VMEM((BM, E), jnp.float32),
            pltpu.VMEM((8, E), jnp.float32),
        ],
        compiler_params=pltpu.CompilerParams(
            dimension_semantics=("arbitrary", "arbitrary"),
        ),
    )(x, W1b, b1r, W2b, b2r)
    return topw, topi, loss.reshape(())
